# baseline (speedup 1.0000x reference)
import functools

import jax
import jax.numpy as jnp
from jax import lax
from jax.experimental import pallas as pl
from jax.experimental.pallas import tpu as pltpu
from jax.experimental.pallas import tpu_sc as plsc

_NC = 2
_NS = 16
_NW = _NC * _NS
_LW = 16
_RC = 128
_TABP = 1024


def _sc_body(ts_hbm, taba_hbm, tabs_hbm, x_hbm, n_hbm, out_hbm,
             ca_v, cs_v, t_v, xb, nb, ob, sem_in, sem_out):
    F = x_hbm.shape[0]
    nchunks = F // _RC
    c = lax.axis_index("c")
    s = lax.axis_index("s")
    wid = s * _NC + c
    c0 = wid * 128

    pltpu.sync_copy(ts_hbm.at[pl.ds(c0, 128)], t_v)
    pltpu.async_copy(taba_hbm.at[t_v], ca_v, sem_in.at[0]).wait()
    pltpu.async_copy(tabs_hbm.at[t_v], cs_v, sem_in.at[0]).wait()

    cas = [ca_v[pl.ds(j * _LW, _LW)] for j in range(8)]
    css = [cs_v[pl.ds(j * _LW, _LW)] for j in range(8)]

    def start_in(g, b):
        r0 = g * _RC
        pltpu.async_copy(x_hbm.at[pl.ds(r0, _RC), pl.ds(c0, 128)],
                         xb.at[b], sem_in.at[b])
        pltpu.async_copy(n_hbm.at[pl.ds(r0, _RC), pl.ds(c0, 128)],
                         nb.at[b], sem_in.at[b])

    start_in(0, 0)
    start_in(1, 1)

    def chunk(g, b):
        pltpu.make_async_copy(x_hbm.at[pl.ds(0, _RC), pl.ds(c0, 128)],
                              xb.at[b], sem_in.at[b]).wait()
        pltpu.make_async_copy(n_hbm.at[pl.ds(0, _RC), pl.ds(c0, 128)],
                              nb.at[b], sem_in.at[b]).wait()
        xbr = xb.at[b]
        nbr = nb.at[b]
        obr = ob.at[b]

        def row(i, carry):
            for j in range(8):
                sl = pl.ds(j * _LW, _LW)
                obr[i, sl] = cas[j] * xbr[i, sl] + css[j] * nbr[i, sl]
            return carry

        lax.fori_loop(0, _RC, row, 0)

        @pl.when(g >= 2)
        def _():
            pltpu.make_async_copy(ob.at[b],
                                  out_hbm.at[pl.ds(0, _RC), pl.ds(c0, 128)],
                                  sem_out.at[b]).wait()

        pltpu.async_copy(ob.at[b], out_hbm.at[pl.ds(g * _RC, _RC),
                                              pl.ds(c0, 128)], sem_out.at[b])

        @pl.when(g + 2 < nchunks)
        def _():
            start_in(g + 2, b)

    def pair(g2, carry):
        chunk(g2 * 2, 0)
        chunk(g2 * 2 + 1, 1)
        return carry

    lax.fori_loop(0, nchunks // 2, pair, 0)

    for b in range(2):
        pltpu.make_async_copy(ob.at[b],
                              out_hbm.at[pl.ds(0, _RC), pl.ds(c0, 128)],
                              sem_out.at[b]).wait()


def kernel(x0, noise, timesteps, sqrt_alphas_cumprod, sqrt_one_minus_alphas_cumprod):
    B, C, H, W = x0.shape
    F = C * H * W
    x = x0.transpose(1, 2, 3, 0).reshape(F, B)
    n = noise.transpose(1, 2, 3, 0).reshape(F, B)
    steps = sqrt_alphas_cumprod.shape[0]
    taba = jnp.zeros((_TABP,), jnp.float32).at[:steps].set(sqrt_alphas_cumprod)
    tabs = jnp.zeros((_TABP,), jnp.float32).at[:steps].set(
        sqrt_one_minus_alphas_cumprod)

    mesh = plsc.VectorSubcoreMesh(core_axis_name="c", subcore_axis_name="s")
    run = functools.partial(
        pl.kernel,
        mesh=mesh,
        out_type=jax.ShapeDtypeStruct((F, B), jnp.float32),
        scratch_types=[
            pltpu.VMEM((128,), jnp.float32),
            pltpu.VMEM((128,), jnp.float32),
            pltpu.VMEM((128,), jnp.int32),
            pltpu.VMEM((2, _RC, 128), jnp.float32),
            pltpu.VMEM((2, _RC, 128), jnp.float32),
            pltpu.VMEM((2, _RC, 128), jnp.float32),
            pltpu.SemaphoreType.DMA((2,)),
            pltpu.SemaphoreType.DMA((2,)),
        ],
    )(_sc_body)
    out = run(timesteps, taba, tabs, x, n)
    return out.reshape(C, H, W, B).transpose(3, 0, 1, 2)


# speedup vs baseline: 1.2007x; 1.2007x over previous
import jax
import jax.numpy as jnp
from jax.experimental import pallas as pl
from jax.experimental.pallas import tpu as pltpu

_STEPS_PAD = 1024
_FB = 256


def _scale_kernel(ts_ref, tab_ref, x_ref, n_ref, o_ref, coef_ref):
    @pl.when(pl.program_id(0) == 0)
    def _():
        t = ts_ref[...]
        B = t.shape[1]
        lo = jnp.bitwise_and(t, 127)
        hi = jnp.right_shift(t, 7)
        onehot = (jax.lax.broadcasted_iota(jnp.int32, (128, B), 0)
                  == lo).astype(jnp.float32)
        d = jax.lax.dot_general(
            tab_ref[...], onehot, (((1,), (0,)), ((), ())),
            precision=jax.lax.Precision.HIGHEST,
            preferred_element_type=jnp.float32)
        ca = jnp.zeros((1, B), jnp.float32)
        cs = jnp.zeros((1, B), jnp.float32)
        for u in range(8):
            m = hi == u
            ca = ca + jnp.where(m, d[u:u + 1, :], 0.0)
            cs = cs + jnp.where(m, d[u + 8:u + 9, :], 0.0)
        coef_ref[0:1, :] = ca
        coef_ref[1:2, :] = cs

    o_ref[...] = (coef_ref[0:1, :] * x_ref[...]
                  + coef_ref[1:2, :] * n_ref[...])


def kernel(x0, noise, timesteps, sqrt_alphas_cumprod, sqrt_one_minus_alphas_cumprod):
    B, C, H, W = x0.shape
    F = C * H * W
    x = x0.transpose(1, 2, 3, 0).reshape(F, B)
    n = noise.transpose(1, 2, 3, 0).reshape(F, B)
    ts2 = timesteps.reshape(1, B)
    steps = sqrt_alphas_cumprod.shape[0]
    tab = jnp.zeros((2, _STEPS_PAD), jnp.float32)
    tab = tab.at[0, :steps].set(sqrt_alphas_cumprod)
    tab = tab.at[1, :steps].set(sqrt_one_minus_alphas_cumprod)
    tab16 = tab.reshape(2, 8, 128).reshape(16, 128)

    out = pl.pallas_call(
        _scale_kernel,
        grid=(F // _FB,),
        in_specs=[
            pl.BlockSpec((1, B), lambda i: (0, 0)),
            pl.BlockSpec((16, 128), lambda i: (0, 0)),
            pl.BlockSpec((_FB, B), lambda i: (i, 0)),
            pl.BlockSpec((_FB, B), lambda i: (i, 0)),
        ],
        out_specs=pl.BlockSpec((_FB, B), lambda i: (i, 0)),
        out_shape=jax.ShapeDtypeStruct((F, B), x0.dtype),
        scratch_shapes=[pltpu.VMEM((2, B), jnp.float32)],
    )(ts2, tab16, x, n)
    return out.reshape(C, H, W, B).transpose(3, 0, 1, 2)
